# speedup vs baseline: 1.3111x; 1.3111x over previous
import functools

import jax
import jax.numpy as jnp
from jax import lax
from jax.experimental import pallas as pl
from jax.experimental.pallas import tpu as pltpu
from jax.experimental.pallas import tpu_sc as plsc

B = 16384
F = 26
NW = 32
ROWS_PER_W = B // NW
GATH_PER_W = ROWS_PER_W * F
CHUNK = 128
NCHUNK = GATH_PER_W // CHUNK
FIRE = 8


def _sc_body(w_hbm, idx_hbm, vals_hbm, b_hbm, out_hbm,
             idx_v, vals_v, rows_v, out_v, b_v, sem):
    cid = lax.axis_index("c")
    sid = lax.axis_index("s")
    wid = sid * 2 + cid

    pltpu.sync_copy(idx_hbm.at[wid], idx_v)
    pltpu.sync_copy(vals_hbm.at[wid], vals_v)
    pltpu.sync_copy(b_hbm, b_v)

    def fire_drain(g, _):
        base = g * FIRE
        copies = [
            pltpu.make_async_copy(
                w_hbm.at[idx_v.at[base + j]], rows_v.at[base + j], sem)
            for j in range(FIRE)
        ]
        for c in copies:
            c.start()
        for c in copies:
            c.wait()
        return _

    lax.fori_loop(0, NCHUNK // FIRE, fire_drain, None, unroll=False)

    def compute_group(t, _):
        c = lax.shift_right_logical(t, 3)
        d = lax.bitwise_and(t, 7)
        col = d * 16
        acc = b_v[...]
        for f in range(F):
            r = f * 4 + c
            acc = acc + rows_v[r, pl.ds(col, 16)] * vals_v[r, pl.ds(col, 16)]
        out_v[pl.ds(c * 128 + col, 16)] = acc
        return _

    lax.fori_loop(0, ROWS_PER_W // 16, compute_group, None, unroll=False)

    pltpu.sync_copy(out_v, out_hbm.at[pl.ds(wid * ROWS_PER_W, ROWS_PER_W)])


@functools.partial(
    pl.kernel,
    out_type=jax.ShapeDtypeStruct((B,), jnp.float32),
    mesh=plsc.VectorSubcoreMesh(core_axis_name="c", subcore_axis_name="s"),
    scratch_types=[
        pltpu.VMEM((NCHUNK, CHUNK), jnp.int32),
        pltpu.VMEM((NCHUNK, CHUNK), jnp.float32),
        pltpu.VMEM((NCHUNK, CHUNK), jnp.float32),
        pltpu.VMEM((ROWS_PER_W,), jnp.float32),
        pltpu.VMEM((16,), jnp.float32),
        pltpu.SemaphoreType.DMA,
    ],
)
def _lr_sc_kernel(w_hbm, idx_hbm, vals_hbm, b_hbm, out_hbm,
                  idx_v, vals_v, rows_v, out_v, b_v, sem):
    _sc_body(w_hbm, idx_hbm, vals_hbm, b_hbm, out_hbm,
             idx_v, vals_v, rows_v, out_v, b_v, sem)


def kernel(indices, values, w, b):
    idx_r = (indices.astype(jnp.int32)
             .reshape(NW, ROWS_PER_W, F)
             .transpose(0, 2, 1)
             .reshape(NW, NCHUNK, CHUNK))
    vals_r = (values
              .reshape(NW, ROWS_PER_W, F)
              .transpose(0, 2, 1)
              .reshape(NW, NCHUNK, CHUNK))
    w_flat = w.reshape(-1)
    b16 = jnp.broadcast_to(b.astype(jnp.float32), (16,))
    out = _lr_sc_kernel(w_flat, idx_r, vals_r, b16)
    return out.reshape(B, 1)


# speedup vs baseline: 1.3763x; 1.0497x over previous
import functools

import jax
import jax.numpy as jnp
from jax import lax
from jax.experimental import pallas as pl
from jax.experimental.pallas import tpu as pltpu
from jax.experimental.pallas import tpu_sc as plsc

B = 16384
F = 26
NW = 32
ROWS_PER_W = B // NW
CHUNK = 128
NCHUNK = ROWS_PER_W * F // CHUNK
NBLK = ROWS_PER_W // CHUNK


def _sc_body(w_hbm, idx_hbm, vals_hbm, b_hbm, out_hbm,
             idx_v, vals_v, rows_v, out_v, b_v, sem0, sem1, vsem):
    cid = lax.axis_index("c")
    sid = lax.axis_index("s")
    wid = sid * 2 + cid
    sems = (sem0, sem1)

    pltpu.sync_copy(idx_hbm.at[wid], idx_v)
    vcopy = pltpu.make_async_copy(vals_hbm.at[wid], vals_v, vsem)
    vcopy.start()
    pltpu.sync_copy(b_hbm, b_v)

    def fire(c):
        cs = []
        for f in range(F):
            r = f * NBLK + c
            cs.append(pltpu.make_async_copy(
                w_hbm.at[idx_v.at[r]], rows_v.at[r], sems[c & 1]))
        for cp in cs:
            cp.start()
        return cs

    pend = fire(0)
    vcopy.wait()
    for c in range(NBLK):
        nxt = fire(c + 1) if c + 1 < NBLK else None
        for cp in pend:
            cp.wait()
        pend = nxt
        for d in range(CHUNK // 16):
            col = d * 16
            acc = b_v[...]
            for f in range(F):
                r = f * NBLK + c
                acc = acc + rows_v[r, pl.ds(col, 16)] * vals_v[r, pl.ds(col, 16)]
            out_v[pl.ds(c * CHUNK + col, 16)] = acc

    pltpu.sync_copy(out_v, out_hbm.at[pl.ds(wid * ROWS_PER_W, ROWS_PER_W)])


@functools.partial(
    pl.kernel,
    out_type=jax.ShapeDtypeStruct((B,), jnp.float32),
    mesh=plsc.VectorSubcoreMesh(core_axis_name="c", subcore_axis_name="s"),
    scratch_types=[
        pltpu.VMEM((NCHUNK, CHUNK), jnp.int32),
        pltpu.VMEM((NCHUNK, CHUNK), jnp.float32),
        pltpu.VMEM((NCHUNK, CHUNK), jnp.float32),
        pltpu.VMEM((ROWS_PER_W,), jnp.float32),
        pltpu.VMEM((16,), jnp.float32),
        pltpu.SemaphoreType.DMA,
        pltpu.SemaphoreType.DMA,
        pltpu.SemaphoreType.DMA,
    ],
)
def _lr_sc_kernel(w_hbm, idx_hbm, vals_hbm, b_hbm, out_hbm,
                  idx_v, vals_v, rows_v, out_v, b_v, sem0, sem1, vsem):
    _sc_body(w_hbm, idx_hbm, vals_hbm, b_hbm, out_hbm,
             idx_v, vals_v, rows_v, out_v, b_v, sem0, sem1, vsem)


def kernel(indices, values, w, b):
    idx_r = (indices.astype(jnp.int32)
             .reshape(NW, ROWS_PER_W, F)
             .transpose(0, 2, 1)
             .reshape(NW, NCHUNK, CHUNK))
    vals_r = (values
              .reshape(NW, ROWS_PER_W, F)
              .transpose(0, 2, 1)
              .reshape(NW, NCHUNK, CHUNK))
    w_flat = w.reshape(-1)
    b16 = jnp.broadcast_to(b.astype(jnp.float32), (16,))
    out = _lr_sc_kernel(w_flat, idx_r, vals_r, b16)
    return out.reshape(B, 1)


# speedup vs baseline: 2.1633x; 1.5718x over previous
import functools

import jax
import jax.numpy as jnp
from jax import lax
from jax.experimental import pallas as pl
from jax.experimental.pallas import tpu as pltpu
from jax.experimental.pallas import tpu_sc as plsc

B = 16384
F = 26
NW = 32
ROWS_PER_W = B // NW
CHUNK = 128
NCHUNK = ROWS_PER_W * F // CHUNK
NBLK = ROWS_PER_W // CHUNK


def _sc_body(w_hbm, idx_hbm, vals_hbm, b_hbm, out_hbm,
             idx_v, vals_v, rows_v, out_v, b_v, sem0, sem1, vsem):
    cid = lax.axis_index("c")
    sid = lax.axis_index("s")
    wid = sid * 2 + cid
    sems = (sem0, sem1)

    pltpu.sync_copy(idx_hbm.at[wid], idx_v)
    vcopy = pltpu.make_async_copy(vals_hbm.at[wid], vals_v, vsem)
    vcopy.start()
    pltpu.sync_copy(b_hbm, b_v)

    def fire(c):
        cs = []
        for f in range(F):
            r = f * NBLK + c
            cs.append(pltpu.make_async_copy(
                w_hbm.at[idx_v.at[r]], rows_v.at[r], sems[c & 1]))
        for cp in cs:
            cp.start()
        return cs

    pend = fire(0)
    vcopy.wait()
    for c in range(NBLK):
        nxt = fire(c + 1) if c + 1 < NBLK else None
        for cp in pend:
            cp.wait()
        pend = nxt
        for d in range(CHUNK // 16):
            col = d * 16
            acc = b_v[...]
            for f in range(F):
                r = f * NBLK + c
                acc = acc + rows_v[r, pl.ds(col, 16)] * vals_v[r, pl.ds(col, 16)]
            out_v[pl.ds(c * CHUNK + col, 16)] = acc

    pltpu.sync_copy(out_v, out_hbm.at[pl.ds(wid * ROWS_PER_W, ROWS_PER_W)])


@functools.partial(
    pl.kernel,
    out_type=jax.ShapeDtypeStruct((B,), jnp.float32),
    mesh=plsc.VectorSubcoreMesh(core_axis_name="c", subcore_axis_name="s"),
    scratch_types=[
        pltpu.VMEM((NCHUNK, CHUNK), jnp.int32),
        pltpu.VMEM((NCHUNK, CHUNK), jnp.float32),
        pltpu.VMEM((NCHUNK, CHUNK), jnp.float32),
        pltpu.VMEM((ROWS_PER_W,), jnp.float32),
        pltpu.VMEM((16,), jnp.float32),
        pltpu.SemaphoreType.DMA,
        pltpu.SemaphoreType.DMA,
        pltpu.SemaphoreType.DMA,
    ],
)
def _lr_sc_kernel(w_hbm, idx_hbm, vals_hbm, b_hbm, out_hbm,
                  idx_v, vals_v, rows_v, out_v, b_v, sem0, sem1, vsem):
    _sc_body(w_hbm, idx_hbm, vals_hbm, b_hbm, out_hbm,
             idx_v, vals_v, rows_v, out_v, b_v, sem0, sem1, vsem)


def kernel(indices, values, w, b):
    idx_r = (indices.astype(jnp.int32)
             .reshape(NW, ROWS_PER_W, F)
             .transpose(0, 2, 1)
             .reshape(NW, NCHUNK, CHUNK))
    vals_r = (values
              .reshape(NW, ROWS_PER_W, F)
              .transpose(0, 2, 1)
              .reshape(NW, NCHUNK, CHUNK))
    w_flat = jax.lax.dynamic_update_slice(jnp.zeros((1000448, 1), jnp.float32), w, (0, 0)).reshape(-1)
    b16 = jnp.broadcast_to(b.astype(jnp.float32), (16,))
    out = _lr_sc_kernel(w_flat, idx_r, vals_r, b16)
    return out.reshape(B, 1)


# speedup vs baseline: 2.4774x; 1.1452x over previous
import functools

import jax
import jax.numpy as jnp
from jax import lax
from jax.experimental import pallas as pl
from jax.experimental.pallas import tpu as pltpu
from jax.experimental.pallas import tpu_sc as plsc

B = 16384
F = 26
NW = 32
ROWS_PER_W = B // NW
SLAB = ROWS_PER_W * F
CHUNK = 128
NCHUNK = SLAB // CHUNK
NBLK = 4
VOCAB = 1000000
SEG_EVEN = 62496
SEG_ODD = 62504
BOUNCE = 12800


def _seg(k):
    base = (k // 2) * (SEG_EVEN + SEG_ODD) + (k % 2) * SEG_EVEN
    return base, (SEG_EVEN if k % 2 == 0 else SEG_ODD)


def _sc_body(w_hbm, idx_hbm, vals_hbm, b_hbm, out_hbm,
             spw, idx_v, vals_v, rows_v, out_v, b_v, sem0, sem1, vsem):
    cid = lax.axis_index("c")
    sid = lax.axis_index("s")
    wid = sid * 2 + cid
    sems = (sem0, sem1)

    icopy = pltpu.make_async_copy(idx_hbm.at[wid], idx_v, vsem)
    icopy.start()
    vcopy = pltpu.make_async_copy(vals_hbm.at[wid], vals_v, vsem)
    vcopy.start()
    pltpu.sync_copy(b_hbm, b_v)

    for k in range(16):
        @pl.when(sid == k)
        def _():
            base, seglen = _seg(k)
            off = 0
            while off < seglen:
                n = min(BOUNCE, seglen - off)
                pltpu.sync_copy(w_hbm.at[pl.ds(base + off, n)],
                                rows_v.at[pl.ds(0, n)])
                pltpu.sync_copy(rows_v.at[pl.ds(0, n)],
                                spw.at[pl.ds(base + off, n)])
                off += n

    icopy.wait()
    vcopy.wait()
    plsc.subcore_barrier()

    def fire(c):
        cs = []
        for f in range(F):
            r = f * NBLK + c
            cs.append(pltpu.make_async_copy(
                spw.at[idx_v.at[pl.ds(r * CHUNK, CHUNK)]],
                rows_v.at[pl.ds(r * CHUNK, CHUNK)], sems[c & 1]))
        for cp in cs:
            cp.start()
        return cs

    pend = fire(0)
    for c in range(NBLK):
        nxt = fire(c + 1) if c + 1 < NBLK else None
        for cp in pend:
            cp.wait()
        pend = nxt
        for d in range(CHUNK // 16):
            col = d * 16
            acc = b_v[...]
            for f in range(F):
                p = (f * NBLK + c) * CHUNK + col
                acc = acc + (rows_v[pl.ds(p, 16)] * vals_v[pl.ds(p, 16)])
            out_v[pl.ds(c * CHUNK + col, 16)] = acc

    pltpu.sync_copy(out_v, out_hbm.at[pl.ds(wid * ROWS_PER_W, ROWS_PER_W)])


@functools.partial(
    pl.kernel,
    out_type=jax.ShapeDtypeStruct((B,), jnp.float32),
    mesh=plsc.VectorSubcoreMesh(core_axis_name="c", subcore_axis_name="s"),
    scratch_types=[
        pltpu.VMEM_SHARED((VOCAB,), jnp.float32),
        pltpu.VMEM((SLAB,), jnp.int32),
        pltpu.VMEM((SLAB,), jnp.float32),
        pltpu.VMEM((SLAB,), jnp.float32),
        pltpu.VMEM((ROWS_PER_W,), jnp.float32),
        pltpu.VMEM((16,), jnp.float32),
        pltpu.SemaphoreType.DMA,
        pltpu.SemaphoreType.DMA,
        pltpu.SemaphoreType.DMA,
    ],
)
def _lr_sc_kernel(w_hbm, idx_hbm, vals_hbm, b_hbm, out_hbm,
                  spw, idx_v, vals_v, rows_v, out_v, b_v, sem0, sem1, vsem):
    _sc_body(w_hbm, idx_hbm, vals_hbm, b_hbm, out_hbm,
             spw, idx_v, vals_v, rows_v, out_v, b_v, sem0, sem1, vsem)


def kernel(indices, values, w, b):
    idx_r = (indices.astype(jnp.int32)
             .reshape(NW, ROWS_PER_W, F)
             .transpose(0, 2, 1)
             .reshape(NW, SLAB))
    vals_r = (values
              .reshape(NW, ROWS_PER_W, F)
              .transpose(0, 2, 1)
              .reshape(NW, SLAB))
    w_flat = jax.lax.dynamic_update_slice(jnp.zeros((1000448, 1), jnp.float32), w, (0, 0)).reshape(-1)
    b16 = jnp.broadcast_to(b.astype(jnp.float32), (16,))
    out = _lr_sc_kernel(w_flat, idx_r, vals_r, b16)
    return out.reshape(B, 1)
